# baseline (speedup 1.0000x reference)
import jax
import jax.numpy as jnp
from jax import lax
from jax.experimental import pallas as pl
from jax.experimental.pallas import tpu as pltpu
from jax.experimental.pallas import tpu_sc as plsc

_C = 10000
_FB = 1000


def _feat_concat(x, gen, N, GN, D):
    XB = N // _FB
    GB = GN // _FB

    def body(x_r, g_r, o_r):
        i = pl.program_id(0)

        @pl.when(i < XB)
        def _():
            o_r[...] = x_r[...]

        @pl.when(i >= XB)
        def _():
            o_r[...] = g_r[...]

    return pl.pallas_call(
        body,
        grid=(XB + GB,),
        in_specs=[
            pl.BlockSpec((_FB, D), lambda i: (jnp.where(i < XB, i, XB - 1), 0)),
            pl.BlockSpec((_FB, D), lambda i: (jnp.where(i < XB, 0, i - XB), 0)),
        ],
        out_specs=pl.BlockSpec((_FB, D), lambda i: (i, 0)),
        out_shape=jax.ShapeDtypeStruct((N + GN, D), jnp.float32),
    )(x, gen)


def kernel(x, edge_index, tails, gen_feat, num_pred):
    N, D = x.shape
    E = edge_index.shape[1]
    T = tails.shape[0]
    P = gen_feat.shape[0] // T
    G = T * P
    W = E + G

    info = plsc.get_sparse_core_info()
    NC, NS = info.num_cores, info.num_subcores
    NW = NC * NS

    CE = 2 * E // (NW * _C)
    GC = max(16, G // NW)
    while G % GC or GC % 16:
        GC += 1
    NACT = G // GC

    mesh = plsc.VectorSubcoreMesh(core_axis_name="c", subcore_axis_name="s")

    def body(edge_h, tails_h, edge_o,
             buf0, buf1, tails_v, rep_v, ids_v, si0, si1, so0, so1):
        bufs = [buf0, buf1]
        sin = [si0, si1]
        sout = [so0, so1]
        wid = lax.axis_index("s") * NC + lax.axis_index("c")

        offs = []
        for j in range(CE):
            o = (wid * CE + j) * _C
            offs.append((o, o + jnp.where(o < E, 0, G)))

        in_h = []
        for j, (so_, _) in enumerate(offs):
            in_h.append(pltpu.async_copy(
                edge_h.at[pl.ds(so_, _C)], bufs[j], sin[j]))

        @pl.when(wid < NACT)
        def _gen():
            pltpu.sync_copy(tails_h, tails_v)
            c0 = wid * GC
            iota = lax.iota(jnp.int32, 16)
            for j in range(GC // 16):
                k = iota + (c0 + j * 16)
                rep_v[pl.ds(j * 16, 16)] = plsc.load_gather(tails_v, [k // P])
                ids_v[pl.ds(j * 16, 16)] = k + N
            pltpu.sync_copy(rep_v, edge_o.at[pl.ds(E + c0, GC)])
            pltpu.sync_copy(ids_v, edge_o.at[pl.ds(W + E + c0, GC)])

        out_h = []
        for j, (_, do_) in enumerate(offs):
            in_h[j].wait()
            out_h.append(pltpu.async_copy(
                bufs[j], edge_o.at[pl.ds(do_, _C)], sout[j]))
        for h in out_h:
            h.wait()

    run = pl.kernel(
        body,
        out_type=[
            jax.ShapeDtypeStruct((2 * W,), jnp.int32),
        ],
        mesh=mesh,
        scratch_types=[
            pltpu.VMEM((_C,), jnp.int32),
            pltpu.VMEM((_C,), jnp.int32),
            pltpu.VMEM((T,), jnp.int32),
            pltpu.VMEM((GC,), jnp.int32),
            pltpu.VMEM((GC,), jnp.int32),
            pltpu.SemaphoreType.DMA,
            pltpu.SemaphoreType.DMA,
            pltpu.SemaphoreType.DMA,
            pltpu.SemaphoreType.DMA,
        ],
        compiler_params=pltpu.CompilerParams(needs_layout_passes=False),
    )

    (edge_flat,) = run(edge_index.reshape(-1), tails)
    new_feat = _feat_concat(
        x, gen_feat.astype(jnp.float32), N, gen_feat.shape[0], D)
    return (new_feat, edge_flat.reshape(2, W))


# speedup vs baseline: 1.2046x; 1.2046x over previous
import jax
import jax.numpy as jnp
from jax import lax
from jax.experimental import pallas as pl
from jax.experimental.pallas import tpu as pltpu
from jax.experimental.pallas import tpu_sc as plsc

_C = 10000
_FB = 5000


def _feat_concat(x, gen, N, GN, D):
    XB = N // _FB
    GB = GN // _FB

    def body(x_r, g_r, o_r):
        i = pl.program_id(0)

        @pl.when(i < XB)
        def _():
            o_r[...] = x_r[...]

        @pl.when(i >= XB)
        def _():
            o_r[...] = g_r[...]

    return pl.pallas_call(
        body,
        grid=(XB + GB,),
        in_specs=[
            pl.BlockSpec((_FB, D), lambda i: (jnp.where(i < XB, i, XB - 1), 0)),
            pl.BlockSpec((_FB, D), lambda i: (jnp.where(i < XB, 0, i - XB), 0)),
        ],
        out_specs=pl.BlockSpec((_FB, D), lambda i: (i, 0)),
        out_shape=jax.ShapeDtypeStruct((N + GN, D), jnp.float32),
    )(x, gen)


def kernel(x, edge_index, tails, gen_feat, num_pred):
    N, D = x.shape
    E = edge_index.shape[1]
    T = tails.shape[0]
    P = gen_feat.shape[0] // T
    G = T * P
    W = E + G

    info = plsc.get_sparse_core_info()
    NC, NS = info.num_cores, info.num_subcores
    NW = NC * NS

    CE = 2 * E // (NW * _C)
    GC = max(16, G // NW)
    while G % GC or GC % 16:
        GC += 1
    NACT = G // GC

    mesh = plsc.VectorSubcoreMesh(core_axis_name="c", subcore_axis_name="s")

    def body(edge_h, tails_h, edge_o,
             buf0, buf1, tails_v, rep_v, ids_v, si0, si1, so0, so1):
        bufs = [buf0, buf1]
        sin = [si0, si1]
        sout = [so0, so1]
        wid = lax.axis_index("s") * NC + lax.axis_index("c")

        offs = []
        for j in range(CE):
            o = (wid * CE + j) * _C
            offs.append((o, o + jnp.where(o < E, 0, G)))

        in_h = []
        for j, (so_, _) in enumerate(offs):
            in_h.append(pltpu.async_copy(
                edge_h.at[pl.ds(so_, _C)], bufs[j], sin[j]))

        @pl.when(wid < NACT)
        def _gen():
            pltpu.sync_copy(tails_h, tails_v)
            c0 = wid * GC
            iota = lax.iota(jnp.int32, 16)
            for j in range(GC // 16):
                k = iota + (c0 + j * 16)
                rep_v[pl.ds(j * 16, 16)] = plsc.load_gather(tails_v, [k // P])
                ids_v[pl.ds(j * 16, 16)] = k + N
            pltpu.sync_copy(rep_v, edge_o.at[pl.ds(E + c0, GC)])
            pltpu.sync_copy(ids_v, edge_o.at[pl.ds(W + E + c0, GC)])

        out_h = []
        for j, (_, do_) in enumerate(offs):
            in_h[j].wait()
            out_h.append(pltpu.async_copy(
                bufs[j], edge_o.at[pl.ds(do_, _C)], sout[j]))
        for h in out_h:
            h.wait()

    run = pl.kernel(
        body,
        out_type=[
            jax.ShapeDtypeStruct((2 * W,), jnp.int32),
        ],
        mesh=mesh,
        scratch_types=[
            pltpu.VMEM((_C,), jnp.int32),
            pltpu.VMEM((_C,), jnp.int32),
            pltpu.VMEM((T,), jnp.int32),
            pltpu.VMEM((GC,), jnp.int32),
            pltpu.VMEM((GC,), jnp.int32),
            pltpu.SemaphoreType.DMA,
            pltpu.SemaphoreType.DMA,
            pltpu.SemaphoreType.DMA,
            pltpu.SemaphoreType.DMA,
        ],
        compiler_params=pltpu.CompilerParams(needs_layout_passes=False),
    )

    (edge_flat,) = run(edge_index.reshape(-1), tails)
    new_feat = _feat_concat(
        x, gen_feat.astype(jnp.float32), N, gen_feat.shape[0], D)
    return (new_feat, edge_flat.reshape(2, W))
